# baseline (speedup 1.0000x reference)
import functools

import jax
import jax.numpy as jnp
from jax import lax
from jax.experimental import pallas as pl
from jax.experimental.pallas import tpu as pltpu
from jax.experimental.pallas import tpu_sc as plsc

B = 4
N = 1024
C = 256
NHEAD = 8
HD = C // NHEAD
NLVL = 4
NPTS = 4
NCLS = 7
H_IMG = 512.0
W_IMG = 512.0
S0 = 64 * 64
S1 = 32 * 32 + 16 * 16 + 8 * 8
LVL1_STARTS = (0, 1024, 1280)
NW = 32
BN = B * N
ROWS = BN * NHEAD
RPW = ROWS // NW
G = 8
GROUPS = RPW // G


def _iota16():
    return lax.iota(jnp.int32, 16)


def _splat(x):
    return jnp.full((16,), x, jnp.int32)


def _floor16(x):
    xi = x.astype(jnp.int32)
    xf = xi.astype(jnp.float32)
    return jnp.where(xf > x, xf - 1.0, xf)


def _mm_body(x_ref, w_ref, b_ref, o_ref):
    o_ref[...] = (
        jnp.dot(x_ref[...], w_ref[...], preferred_element_type=jnp.float32)
        + b_ref[...]
    ).astype(o_ref.dtype)


def _mm(x, w, b, bm=512, out_dtype=jnp.float32):
    m, k = x.shape
    kn = w.shape[1]
    return pl.pallas_call(
        _mm_body,
        grid=(m // bm,),
        in_specs=[
            pl.BlockSpec((bm, k), lambda i: (i, 0)),
            pl.BlockSpec((k, kn), lambda i: (0, 0)),
            pl.BlockSpec((1, kn), lambda i: (0, 0)),
        ],
        out_specs=pl.BlockSpec((bm, kn), lambda i: (i, 0)),
        out_shape=jax.ShapeDtypeStruct((m, kn), out_dtype),
    )(x, w, b.reshape(1, kn))


def _pool_body(v0_ref, t1_ref, idx_v, l1_v, d2_v, d3_v, sem):
    wid = lax.axis_index("s") * 2 + lax.axis_index("c")
    iota = _iota16()
    b0 = wid // 8
    w8 = wid % 8

    copies = []
    for corner in range(4):
        dy, dx = corner // 2, corner % 2

        def fill1(i, _):
            q = _splat(wid * 128 + i * 16) + iota
            b = q >> 10
            r = q & 1023
            y = r >> 5
            x = r & 31
            src = (b << 12) + ((y * 2 + dy) << 6) + (x * 2 + dx)
            idx_v[corner, pl.ds(i * 16, 16)] = src
            return 0

        lax.fori_loop(0, 8, fill1, 0)
        c = pltpu.async_copy(
            v0_ref.at[idx_v.at[corner]], l1_v, sem, add=(corner > 0)
        )
        if corner == 0:
            c.wait()
        else:
            copies.append(c)
    for c in copies:
        c.wait()
    pltpu.sync_copy(l1_v, t1_ref.at[b0, pl.ds(w8 * 128, 128)])

    def lvl2(i, _):
        y2 = i >> 4
        x2 = i & 15
        s00 = (y2 * 2) * 32 + x2 * 2
        s10 = (y2 * 2 + 1) * 32 + x2 * 2
        for cc in range(8):
            acc = (l1_v[s00, pl.ds(cc * 32, 32)]
                   + l1_v[s00 + 1, pl.ds(cc * 32, 32)]
                   + l1_v[s10, pl.ds(cc * 32, 32)]
                   + l1_v[s10 + 1, pl.ds(cc * 32, 32)])
            d2_v[i, pl.ds(cc * 32, 32)] = acc
        return 0

    lax.fori_loop(0, 32, lvl2, 0)
    pltpu.sync_copy(d2_v, t1_ref.at[b0, pl.ds(1024 + w8 * 32, 32)])

    def lvl3(i, _):
        for cc in range(8):
            acc = jnp.zeros((32,), jnp.bfloat16)
            for dy in range(4):
                for dx in range(4):
                    acc = acc + l1_v[dy * 32 + i * 4 + dx, pl.ds(cc * 32, 32)]
            d3_v[i, pl.ds(cc * 32, 32)] = acc
        return 0

    lax.fori_loop(0, 8, lvl3, 0)
    pltpu.sync_copy(d3_v, t1_ref.at[b0, pl.ds(1280 + w8 * 8, 8)])


def _pool(v0):
    mesh = plsc.VectorSubcoreMesh(core_axis_name="c", subcore_axis_name="s")
    return pl.kernel(
        _pool_body,
        compiler_params=pltpu.CompilerParams(use_tc_tiling_on_sc=False, needs_layout_passes=False),
        out_type=jax.ShapeDtypeStruct((B, S1, C), jnp.bfloat16),
        mesh=mesh,
        scratch_types=[
            pltpu.VMEM((4, 128), jnp.int32),
            pltpu.VMEM((128, C), jnp.bfloat16),
            pltpu.VMEM((32, C), jnp.bfloat16),
            pltpu.VMEM((8, C), jnp.bfloat16),
            pltpu.SemaphoreType.DMA,
        ],
    )(v0)


def _sample_body(t0_ref, t1_ref, oa_ref, p_ref, out_ref,
                 oa_v, p_v, attw_v, idx0_v, idx1_v, w_v,
                 rows0_v, rows1_v, out_v, sem0, sem1):
    wid = lax.axis_index("s") * 2 + lax.axis_index("c")
    iota = _iota16()
    pidx = iota >> 2
    cx = iota & 1
    cy = (iota >> 1) & 1
    cxf = cx.astype(jnp.float32)
    cyf = cy.astype(jnp.float32)

    bn0 = wid * 128
    pltpu.sync_copy(oa_ref.at[pl.ds(bn0 * 384, 128 * 384)], oa_v)
    pltpu.sync_copy(p_ref.at[pl.ds(bn0 * 2, 128 * 2)], p_v)

    b = wid // 8

    def a_head(grp, par, h, px, py):
        att16 = oa_v[pl.ds(grp * 384 + C + h * 16, 16)]
        m = lax.reduce_max(att16, (0,))
        e = jnp.exp(att16 - m)
        s = lax.reduce_sum(e, (0,))
        attw_v[...] = e / s
        for l in range(NLVL):
            wl = 64 >> l
            att_rep = plsc.load_gather(attw_v, [l * 4 + pidx])
            obase = _splat(grp * 384 + h * 32 + l * 8) + pidx * 2
            offx = plsc.load_gather(oa_v, [obase])
            offy = plsc.load_gather(oa_v, [obase + 1])
            vx = px * float(wl) + offx - 0.5
            vy = py * float(wl) + offy - 0.5
            x0f = _floor16(vx)
            y0f = _floor16(vy)
            wxf = vx - x0f
            wyf = vy - y0f
            ix = x0f.astype(jnp.int32) + cx
            iy = y0f.astype(jnp.int32) + cy
            wxc = cxf * wxf + (1.0 - cxf) * (1.0 - wxf)
            wyc = cyf * wyf + (1.0 - cyf) * (1.0 - wyf)
            valid = ((ix >= 0) & (ix < wl) & (iy >= 0) & (iy < wl))
            scale = 1.0 / float(4 ** l)
            w = jnp.where(valid, att_rep * wxc * wyc * scale, 0.0)
            ixc = jnp.clip(ix, 0, wl - 1)
            iyc = jnp.clip(iy, 0, wl - 1)
            lin = iyc * wl + ixc
            if l == 0:
                rowid = ((b * S0 + lin) << 3) + h
                idx0_v[par, pl.ds(h * 16, 16)] = rowid
            else:
                rowid = ((b * S1 + LVL1_STARTS[l - 1] + lin) << 3) + h
                idx1_v[par, pl.ds(h * 48 + (l - 1) * 16, 16)] = rowid
            w_v[pl.ds(par * 512 + h * 64 + l * 16, 16)] = w

    def b_head(par, h):
        z = jnp.zeros((16,), jnp.float32)
        acc = [z, z, z, z]
        for j in range(16):
            w = plsc.load_gather(w_v, [_splat(par * 512 + h * 64 + j)])
            pair = rows0_v[par * 128 + h * 16 + j, pl.ds(0, 32)]
            ev, od = plsc.unpack(pair, format=plsc.PackFormat.INTERLEAVED)
            acc[j % 2] = acc[j % 2] + w * ev
            acc[2 + j % 2] = acc[2 + j % 2] + w * od
        for j in range(48):
            w = plsc.load_gather(w_v, [_splat(par * 512 + h * 64 + 16 + j)])
            pair = rows1_v[par * 384 + h * 48 + j, pl.ds(0, 32)]
            ev, od = plsc.unpack(pair, format=plsc.PackFormat.INTERLEAVED)
            acc[j % 2] = acc[j % 2] + w * ev
            acc[2 + j % 2] = acc[2 + j % 2] + w * od
        base = _splat(par * 256 + h * 32) + iota * 2
        plsc.store_scatter(out_v, [base], acc[0] + acc[1])
        plsc.store_scatter(out_v, [base + 1], acc[2] + acc[3])

    def phase_a(grp, par):
        px = plsc.load_gather(p_v, [_splat(grp * 2)])
        py = plsc.load_gather(p_v, [_splat(grp * 2 + 1)])
        px = jnp.clip(px * (1.0 / W_IMG), 0.0, 1.0)
        py = jnp.clip(py * (1.0 / H_IMG), 0.0, 1.0)

        def head_body(h, _):
            a_head(grp, par, h, px, py)
            return 0

        lax.fori_loop(0, G, head_body, 0)

    def _copies(par, sem):
        cps = [pltpu.make_async_copy(
            t0_ref.at[idx0_v.at[par]],
            rows0_v.at[pl.ds(par * 128, 128)], sem)]
        cps += [pltpu.make_async_copy(
            t1_ref.at[idx1_v.at[par, pl.ds(k * 128, 128)]],
            rows1_v.at[pl.ds(par * 384 + k * 128, 128)], sem)
            for k in range(3)]
        return cps

    def fire(par, sem):
        for c in _copies(par, sem):
            c.start()

    def drain(par, sem):
        for c in _copies(par, sem):
            c.wait()

    phase_a(0, 0)
    fire(0, sem0)
    phase_a(1, 1)
    fire(1, sem1)

    def pair_body(t, _):
        for par, sem in ((0, sem0), (1, sem1)):
            k = 2 * t + par
            ga = jnp.minimum(k + 2, GROUPS - 1)
            drain(par, sem)
            px = plsc.load_gather(p_v, [_splat(ga * 2)])
            py = plsc.load_gather(p_v, [_splat(ga * 2 + 1)])
            px = jnp.clip(px * (1.0 / W_IMG), 0.0, 1.0)
            py = jnp.clip(py * (1.0 / H_IMG), 0.0, 1.0)

            def fused_head(h, _):
                b_head(par, h)
                a_head(ga, par, h, px, py)
                return 0

            lax.fori_loop(0, G, fused_head, 0)
            pltpu.sync_copy(
                out_v.at[pl.ds(par * 256, 256)],
                out_ref.at[pl.ds((wid * RPW + k * G) * HD, G * HD)],
            )

            @pl.when(k < GROUPS - 2)
            def _():
                fire(par, sem)
        return 0

    lax.fori_loop(0, GROUPS // 2, pair_body, 0)


def _sample(t0, t1, oa, p):
    mesh = plsc.VectorSubcoreMesh(core_axis_name="c", subcore_axis_name="s")
    return pl.kernel(
        _sample_body,
        compiler_params=pltpu.CompilerParams(use_tc_tiling_on_sc=False, needs_layout_passes=False),
        out_type=jax.ShapeDtypeStruct((ROWS * HD,), jnp.float32),
        mesh=mesh,
        scratch_types=[
            pltpu.VMEM((128 * 384,), jnp.float32),
            pltpu.VMEM((128 * 2,), jnp.float32),
            pltpu.VMEM((16,), jnp.float32),
            pltpu.VMEM((2, G * 16), jnp.int32),
            pltpu.VMEM((2, G * 48), jnp.int32),
            pltpu.VMEM((2 * G * 64,), jnp.float32),
            pltpu.VMEM((2 * G * 16, HD), jnp.bfloat16),
            pltpu.VMEM((2 * G * 48, HD), jnp.bfloat16),
            pltpu.VMEM((2 * G * HD,), jnp.float32),
            pltpu.SemaphoreType.DMA,
            pltpu.SemaphoreType.DMA,
        ],
    )(t0, t1, oa, p)


def _head_body(q_ref, a_ref, wo_ref, bo_ref, g_ref, be_ref,
               w1a_ref, w1b_ref, b1_ref, w2_ref, b2_ref, o_ref):
    q = q_ref[...]
    y = q + jnp.dot(a_ref[...], wo_ref[...],
                    preferred_element_type=jnp.float32) + bo_ref[...]
    mu = jnp.mean(y, axis=1, keepdims=True)
    d = y - mu
    var = jnp.mean(d * d, axis=1, keepdims=True)
    ql = d * jax.lax.rsqrt(var + 1e-5) * g_ref[...] + be_ref[...]
    h1 = jnp.dot(q, w1a_ref[...], preferred_element_type=jnp.float32)
    h1 = h1 + jnp.dot(ql, w1b_ref[...], preferred_element_type=jnp.float32)
    h1 = jnp.maximum(h1 + b1_ref[...], 0.0)
    o_ref[...] = jnp.dot(h1, w2_ref[...],
                         preferred_element_type=jnp.float32) + b2_ref[...]


def _head(q, a, wo, bo, g, be, w1a, w1b, b1, w2p, b2p, bm=512):
    args = (q, a, wo, bo.reshape(1, C), g.reshape(1, C), be.reshape(1, C),
            w1a, w1b, b1.reshape(1, C), w2p, b2p.reshape(1, 128))
    blk = pl.BlockSpec((bm, C), lambda i: (i, 0))
    full = lambda v: pl.BlockSpec(v.shape, lambda i: (0,) * v.ndim)
    return pl.pallas_call(
        _head_body,
        grid=(BN // bm,),
        in_specs=[blk, blk] + [full(v) for v in args[2:]],
        out_specs=pl.BlockSpec((bm, 128), lambda i: (i, 0)),
        out_shape=jax.ShapeDtypeStruct((BN, 128), jnp.float32),
    )(*args)


def kernel(q_t, h_t, p_head_t, W_value, b_value, W_off, b_off, W_attn, b_attn,
           W_out, b_out, ln_g, ln_b, W1, b1, W2, b2):
    q_flat = q_t.reshape(BN, C)
    v0 = _mm(h_t.reshape(B * S0, C), W_value, b_value,
             out_dtype=jnp.bfloat16)
    wcat = jnp.concatenate([W_off, W_attn], axis=1)
    bcat = jnp.concatenate([b_off, b_attn], axis=0)
    offatt = _mm(q_flat, wcat, bcat)

    t1 = _pool(v0)
    attn_flat = _sample(
        v0.reshape(B * S0 * NHEAD, HD),
        t1.reshape(B * S1 * NHEAD, HD),
        offatt.reshape(BN * 384), p_head_t.reshape(BN * 2),
    ).reshape(BN, C)

    w2p = jnp.zeros((C, 128), jnp.float32).at[:, :NCLS].set(W2)
    b2p = jnp.zeros((128,), jnp.float32).at[:NCLS].set(b2)
    out = _head(q_flat, attn_flat, W_out, b_out, ln_g, ln_b,
                W1[:C], W1[C:], b1, w2p, b2p)
    return out[:, :NCLS].reshape(B, N, NCLS)


# speedup vs baseline: 1.1061x; 1.1061x over previous
import functools

import jax
import jax.numpy as jnp
from jax import lax
from jax.experimental import pallas as pl
from jax.experimental.pallas import tpu as pltpu
from jax.experimental.pallas import tpu_sc as plsc

B = 4
N = 1024
C = 256
NHEAD = 8
HD = C // NHEAD
NLVL = 4
NPTS = 4
NCLS = 7
H_IMG = 512.0
W_IMG = 512.0
S0 = 64 * 64
S1 = 32 * 32 + 16 * 16 + 8 * 8
LVL1_STARTS = (0, 1024, 1280)
NW = 32
BN = B * N
ROWS = BN * NHEAD
RPW = ROWS // NW
G = 8
GROUPS = RPW // G


def _iota16():
    return lax.iota(jnp.int32, 16)


def _splat(x):
    return jnp.full((16,), x, jnp.int32)


def _floor16(x):
    xi = x.astype(jnp.int32)
    xf = xi.astype(jnp.float32)
    return jnp.where(xf > x, xf - 1.0, xf)


def _mm_body(x_ref, w_ref, b_ref, o_ref):
    o_ref[...] = (
        jnp.dot(x_ref[...], w_ref[...], preferred_element_type=jnp.float32)
        + b_ref[...]
    ).astype(o_ref.dtype)


def _mm(x, w, b, bm=512, out_dtype=jnp.float32):
    m, k = x.shape
    kn = w.shape[1]
    return pl.pallas_call(
        _mm_body,
        grid=(m // bm,),
        in_specs=[
            pl.BlockSpec((bm, k), lambda i: (i, 0)),
            pl.BlockSpec((k, kn), lambda i: (0, 0)),
            pl.BlockSpec((1, kn), lambda i: (0, 0)),
        ],
        out_specs=pl.BlockSpec((bm, kn), lambda i: (i, 0)),
        out_shape=jax.ShapeDtypeStruct((m, kn), out_dtype),
    )(x, w, b.reshape(1, kn))


def _pool_body(v0_ref, t1_ref, idx_v, l1_v, d2_v, d3_v, sem):
    wid = lax.axis_index("s") * 2 + lax.axis_index("c")
    iota = _iota16()
    b0 = wid // 8
    w8 = wid % 8

    copies = []
    for corner in range(4):
        dy, dx = corner // 2, corner % 2

        def fill1(i, _):
            q = _splat(wid * 128 + i * 16) + iota
            b = q >> 10
            r = q & 1023
            y = r >> 5
            x = r & 31
            src = (b << 12) + ((y * 2 + dy) << 6) + (x * 2 + dx)
            idx_v[corner, pl.ds(i * 16, 16)] = src
            return 0

        lax.fori_loop(0, 8, fill1, 0)
        c = pltpu.async_copy(
            v0_ref.at[idx_v.at[corner]], l1_v, sem, add=(corner > 0)
        )
        if corner == 0:
            c.wait()
        else:
            copies.append(c)
    for c in copies:
        c.wait()
    pltpu.sync_copy(l1_v, t1_ref.at[b0, pl.ds(w8 * 128, 128)])

    def lvl2(i, _):
        y2 = i >> 4
        x2 = i & 15
        s00 = (y2 * 2) * 32 + x2 * 2
        s10 = (y2 * 2 + 1) * 32 + x2 * 2
        for cc in range(16):
            acc = (l1_v[s00, pl.ds(cc * 16, 16)]
                   + l1_v[s00 + 1, pl.ds(cc * 16, 16)]
                   + l1_v[s10, pl.ds(cc * 16, 16)]
                   + l1_v[s10 + 1, pl.ds(cc * 16, 16)])
            d2_v[i, pl.ds(cc * 16, 16)] = acc
        return 0

    lax.fori_loop(0, 32, lvl2, 0)
    pltpu.sync_copy(d2_v, t1_ref.at[b0, pl.ds(1024 + w8 * 32, 32)])

    def lvl3(i, _):
        for cc in range(16):
            acc = jnp.zeros((16,), jnp.float32)
            for dy in range(4):
                for dx in range(4):
                    acc = acc + l1_v[dy * 32 + i * 4 + dx, pl.ds(cc * 16, 16)]
            d3_v[i, pl.ds(cc * 16, 16)] = acc
        return 0

    lax.fori_loop(0, 8, lvl3, 0)
    pltpu.sync_copy(d3_v, t1_ref.at[b0, pl.ds(1280 + w8 * 8, 8)])


def _pool(v0):
    mesh = plsc.VectorSubcoreMesh(core_axis_name="c", subcore_axis_name="s")
    return pl.kernel(
        _pool_body,
        compiler_params=pltpu.CompilerParams(use_tc_tiling_on_sc=False, needs_layout_passes=False),
        out_type=jax.ShapeDtypeStruct((B, S1, C), jnp.float32),
        mesh=mesh,
        scratch_types=[
            pltpu.VMEM((4, 128), jnp.int32),
            pltpu.VMEM((128, C), jnp.float32),
            pltpu.VMEM((32, C), jnp.float32),
            pltpu.VMEM((8, C), jnp.float32),
            pltpu.SemaphoreType.DMA,
        ],
    )(v0)


def _sample_body(t0_ref, t1_ref, oa_ref, p_ref, out_ref,
                 oa_v, p_v, attw_v, idx0_v, idx1_v, w_v,
                 rows0_v, rows1_v, out_v, sem0, sem1, osem0, osem1):
    wid = lax.axis_index("s") * 2 + lax.axis_index("c")
    iota = _iota16()
    pidx = iota >> 2
    cx = iota & 1
    cy = (iota >> 1) & 1
    cxf = cx.astype(jnp.float32)
    cyf = cy.astype(jnp.float32)

    bn0 = wid * 128
    pltpu.sync_copy(oa_ref.at[pl.ds(bn0 * 384, 128 * 384)], oa_v)
    pltpu.sync_copy(p_ref.at[pl.ds(bn0 * 2, 128 * 2)], p_v)

    b = wid // 8

    def a_head(grp, par, h, px, py):
        att16 = oa_v[pl.ds(grp * 384 + C + h * 16, 16)]
        m = lax.reduce_max(att16, (0,))
        e = jnp.exp(att16 - m)
        s = lax.reduce_sum(e, (0,))
        attw_v[...] = e / s
        for l in range(NLVL):
            wl = 64 >> l
            att_rep = plsc.load_gather(attw_v, [l * 4 + pidx])
            obase = _splat(grp * 384 + h * 32 + l * 8) + pidx * 2
            offx = plsc.load_gather(oa_v, [obase])
            offy = plsc.load_gather(oa_v, [obase + 1])
            vx = px * float(wl) + offx - 0.5
            vy = py * float(wl) + offy - 0.5
            x0f = _floor16(vx)
            y0f = _floor16(vy)
            wxf = vx - x0f
            wyf = vy - y0f
            ix = x0f.astype(jnp.int32) + cx
            iy = y0f.astype(jnp.int32) + cy
            wxc = cxf * wxf + (1.0 - cxf) * (1.0 - wxf)
            wyc = cyf * wyf + (1.0 - cyf) * (1.0 - wyf)
            valid = ((ix >= 0) & (ix < wl) & (iy >= 0) & (iy < wl))
            scale = 1.0 / float(4 ** l)
            w = jnp.where(valid, att_rep * wxc * wyc * scale, 0.0)
            ixc = jnp.clip(ix, 0, wl - 1)
            iyc = jnp.clip(iy, 0, wl - 1)
            lin = iyc * wl + ixc
            if l == 0:
                rowid = ((b * S0 + lin) << 3) + h
                idx0_v[par, pl.ds(h * 16, 16)] = rowid
            else:
                rowid = ((b * S1 + LVL1_STARTS[l - 1] + lin) << 3) + h
                idx1_v[par, pl.ds(h * 48 + (l - 1) * 16, 16)] = rowid
            w_v[pl.ds(par * 512 + h * 64 + l * 16, 16)] = w

    def b_head(par, h):
        z = jnp.zeros((16,), jnp.float32)
        acc = [z, z, z, z]
        for j in range(16):
            w = plsc.load_gather(w_v, [_splat(par * 512 + h * 64 + j)])
            r = par * 128 + h * 16 + j
            acc[j % 2] = acc[j % 2] + w * rows0_v[r, pl.ds(0, 16)]
            acc[2 + j % 2] = acc[2 + j % 2] + w * rows0_v[r, pl.ds(16, 16)]
        for j in range(48):
            w = plsc.load_gather(w_v, [_splat(par * 512 + h * 64 + 16 + j)])
            r = par * 384 + h * 48 + j
            acc[j % 2] = acc[j % 2] + w * rows1_v[r, pl.ds(0, 16)]
            acc[2 + j % 2] = acc[2 + j % 2] + w * rows1_v[r, pl.ds(16, 16)]
        out_v[pl.ds(par * 256 + h * 32, 16)] = acc[0] + acc[1]
        out_v[pl.ds(par * 256 + h * 32 + 16, 16)] = acc[2] + acc[3]

    def phase_a(grp, par):
        px = plsc.load_gather(p_v, [_splat(grp * 2)])
        py = plsc.load_gather(p_v, [_splat(grp * 2 + 1)])
        px = jnp.clip(px * (1.0 / W_IMG), 0.0, 1.0)
        py = jnp.clip(py * (1.0 / H_IMG), 0.0, 1.0)

        def head_body(h, _):
            a_head(grp, par, h, px, py)
            return 0

        lax.fori_loop(0, G, head_body, 0)

    def _copies(par, sem):
        cps = [pltpu.make_async_copy(
            t0_ref.at[idx0_v.at[par]],
            rows0_v.at[pl.ds(par * 128, 128)], sem)]
        cps += [pltpu.make_async_copy(
            t1_ref.at[idx1_v.at[par, pl.ds(k * 128, 128)]],
            rows1_v.at[pl.ds(par * 384 + k * 128, 128)], sem)
            for k in range(3)]
        return cps

    def fire(par, sem):
        for c in _copies(par, sem):
            c.start()

    def drain(par, sem):
        for c in _copies(par, sem):
            c.wait()

    phase_a(0, 0)
    fire(0, sem0)
    phase_a(1, 1)
    fire(1, sem1)

    def pair_body(t, _):
        for par, sem in ((0, sem0), (1, sem1)):
            k = 2 * t + par
            ga = jnp.minimum(k + 2, GROUPS - 1)
            drain(par, sem)
            px = plsc.load_gather(p_v, [_splat(ga * 2)])
            py = plsc.load_gather(p_v, [_splat(ga * 2 + 1)])
            px = jnp.clip(px * (1.0 / W_IMG), 0.0, 1.0)
            py = jnp.clip(py * (1.0 / H_IMG), 0.0, 1.0)

            def fused_head(h, _):
                b_head(par, h)
                a_head(ga, par, h, px, py)
                return 0

            osem = osem0 if par == 0 else osem1

            @pl.when(k >= 2)
            def _():
                pltpu.make_async_copy(
                    out_v.at[pl.ds(par * 256, 256)],
                    out_ref.at[pl.ds((wid * RPW + (k - 2) * G) * HD, G * HD)],
                    osem,
                ).wait()

            lax.fori_loop(0, G, fused_head, 0)
            pltpu.make_async_copy(
                out_v.at[pl.ds(par * 256, 256)],
                out_ref.at[pl.ds((wid * RPW + k * G) * HD, G * HD)],
                osem,
            ).start()

            @pl.when(k < GROUPS - 2)
            def _():
                fire(par, sem)
        return 0

    lax.fori_loop(0, GROUPS // 2, pair_body, 0)
    pltpu.make_async_copy(
        out_v.at[pl.ds(0, 256)],
        out_ref.at[pl.ds((wid * RPW + (GROUPS - 2) * G) * HD, G * HD)],
        osem0,
    ).wait()
    pltpu.make_async_copy(
        out_v.at[pl.ds(256, 256)],
        out_ref.at[pl.ds((wid * RPW + (GROUPS - 1) * G) * HD, G * HD)],
        osem1,
    ).wait()


def _sample(t0, t1, oa, p):
    mesh = plsc.VectorSubcoreMesh(core_axis_name="c", subcore_axis_name="s")
    return pl.kernel(
        _sample_body,
        compiler_params=pltpu.CompilerParams(use_tc_tiling_on_sc=False, needs_layout_passes=False),
        out_type=jax.ShapeDtypeStruct((ROWS * HD,), jnp.float32),
        mesh=mesh,
        scratch_types=[
            pltpu.VMEM((128 * 384,), jnp.float32),
            pltpu.VMEM((128 * 2,), jnp.float32),
            pltpu.VMEM((16,), jnp.float32),
            pltpu.VMEM((2, G * 16), jnp.int32),
            pltpu.VMEM((2, G * 48), jnp.int32),
            pltpu.VMEM((2 * G * 64,), jnp.float32),
            pltpu.VMEM((2 * G * 16, HD), jnp.float32),
            pltpu.VMEM((2 * G * 48, HD), jnp.float32),
            pltpu.VMEM((2 * G * HD,), jnp.float32),
            pltpu.SemaphoreType.DMA,
            pltpu.SemaphoreType.DMA,
            pltpu.SemaphoreType.DMA,
            pltpu.SemaphoreType.DMA,
        ],
    )(t0, t1, oa, p)


def _head_body(q_ref, a_ref, wo_ref, bo_ref, g_ref, be_ref,
               w1a_ref, w1b_ref, b1_ref, w2_ref, b2_ref, o_ref):
    q = q_ref[...]
    y = q + jnp.dot(a_ref[...], wo_ref[...],
                    preferred_element_type=jnp.float32) + bo_ref[...]
    mu = jnp.mean(y, axis=1, keepdims=True)
    d = y - mu
    var = jnp.mean(d * d, axis=1, keepdims=True)
    ql = d * jax.lax.rsqrt(var + 1e-5) * g_ref[...] + be_ref[...]
    h1 = jnp.dot(q, w1a_ref[...], preferred_element_type=jnp.float32)
    h1 = h1 + jnp.dot(ql, w1b_ref[...], preferred_element_type=jnp.float32)
    h1 = jnp.maximum(h1 + b1_ref[...], 0.0)
    o_ref[...] = jnp.dot(h1, w2_ref[...],
                         preferred_element_type=jnp.float32) + b2_ref[...]


def _head(q, a, wo, bo, g, be, w1a, w1b, b1, w2p, b2p, bm=512):
    args = (q, a, wo, bo.reshape(1, C), g.reshape(1, C), be.reshape(1, C),
            w1a, w1b, b1.reshape(1, C), w2p, b2p.reshape(1, 128))
    blk = pl.BlockSpec((bm, C), lambda i: (i, 0))
    full = lambda v: pl.BlockSpec(v.shape, lambda i: (0,) * v.ndim)
    return pl.pallas_call(
        _head_body,
        grid=(BN // bm,),
        in_specs=[blk, blk] + [full(v) for v in args[2:]],
        out_specs=pl.BlockSpec((bm, 128), lambda i: (i, 0)),
        out_shape=jax.ShapeDtypeStruct((BN, 128), jnp.float32),
    )(*args)


def kernel(q_t, h_t, p_head_t, W_value, b_value, W_off, b_off, W_attn, b_attn,
           W_out, b_out, ln_g, ln_b, W1, b1, W2, b2):
    q_flat = q_t.reshape(BN, C)
    v0 = _mm(h_t.reshape(B * S0, C), W_value, b_value)
    wcat = jnp.concatenate([W_off, W_attn], axis=1)
    bcat = jnp.concatenate([b_off, b_attn], axis=0)
    offatt = _mm(q_flat, wcat, bcat)

    t1 = _pool(v0)
    attn_flat = _sample(
        v0.reshape(B * S0 * NHEAD, HD),
        t1.reshape(B * S1 * NHEAD, HD),
        offatt.reshape(BN * 384), p_head_t.reshape(BN * 2),
    ).reshape(BN, C)

    w2p = jnp.zeros((C, 128), jnp.float32).at[:, :NCLS].set(W2)
    b2p = jnp.zeros((128,), jnp.float32).at[:NCLS].set(b2)
    out = _head(q_flat, attn_flat, W_out, b_out, ln_g, ln_b,
                W1[:C], W1[C:], b1, w2p, b2p)
    return out[:, :NCLS].reshape(B, N, NCLS)
